# baseline (speedup 1.0000x reference)
import jax
import jax.numpy as jnp
from jax.experimental import pallas as pl
from jax.experimental.pallas import tpu as pltpu

_EPS = 1e-5
_BM = 200


def _pass1_kernel(x_ref, w1_ref, adj_a_ref, adj_b_ref, s1_ref, sh1_ref,
                  w2_ref, ta_ref, tb_ref, support_ref):
    @pl.when(pl.program_id(0) == 0)
    def _():
        support_ref[...] = jnp.dot(
            x_ref[...].astype(jnp.bfloat16), w1_ref[...],
            preferred_element_type=jnp.float32).astype(jnp.bfloat16)

    for adj_ref, t_ref in ((adj_a_ref, ta_ref), (adj_b_ref, tb_ref)):
        acc = jnp.dot(adj_ref[...].astype(jnp.bfloat16), support_ref[...],
                      preferred_element_type=jnp.float32)
        h1 = jnp.maximum(acc * s1_ref[...] + sh1_ref[...], 0.0)
        t_ref[...] = jnp.dot(
            h1.astype(jnp.bfloat16), w2_ref[...],
            preferred_element_type=jnp.float32).astype(jnp.bfloat16)


def _pass2_kernel(adj_a_ref, adj_b_ref, ta_ref, tb_ref, s2_ref, sh2_ref,
                  dw_ref, db_ref, mua_ref, mub_ref, outa_ref, outb_ref):
    half = ta_ref.shape[0]
    for adj_ref, mu_ref, out_ref in ((adj_a_ref, mua_ref, outa_ref),
                                     (adj_b_ref, mub_ref, outb_ref)):
        adj16 = adj_ref[...].astype(jnp.bfloat16)
        acc = (jnp.dot(adj16[:, :half], ta_ref[...],
                       preferred_element_type=jnp.float32)
               + jnp.dot(adj16[:, half:], tb_ref[...],
                         preferred_element_type=jnp.float32))
        mu = acc * s2_ref[...] + sh2_ref[...]
        mu_ref[...] = mu
        out_ref[...] = jnp.dot(mu, dw_ref[...],
                               preferred_element_type=jnp.float32) + db_ref[...]


def kernel(x, adj, W1, W2, g1, b1, m1, v1, g2, b2, m2, v2, dec_W, dec_b):
    N, F = x.shape
    H1 = W1.shape[1]
    H2 = W2.shape[1]
    C = dec_W.shape[0]

    inv1 = g1 / jnp.sqrt(v1 + _EPS)
    s1 = inv1.reshape(1, H1)
    sh1 = (b1 - m1 * inv1).reshape(1, H1)
    inv2 = g2 / jnp.sqrt(v2 + _EPS)
    s2 = inv2.reshape(1, H2)
    sh2 = (b2 - m2 * inv2).reshape(1, H2)
    dwT = dec_W.T
    db = dec_b.reshape(1, C)

    BM = _BM
    nblk = N // (2 * BM)
    grid = (nblk,)
    half = N // 2
    const = lambda i: (0, 0)
    top = lambda i: (i, 0)
    bot = lambda i: (i + nblk, 0)

    ta, tb = pl.pallas_call(
        _pass1_kernel,
        grid=grid,
        in_specs=[
            pl.BlockSpec((N, F), const),
            pl.BlockSpec((F, H1), const),
            pl.BlockSpec((BM, N), top),
            pl.BlockSpec((BM, N), bot),
            pl.BlockSpec((1, H1), const),
            pl.BlockSpec((1, H1), const),
            pl.BlockSpec((H1, H2), const),
        ],
        out_specs=[
            pl.BlockSpec((BM, H2), top),
            pl.BlockSpec((BM, H2), top),
        ],
        out_shape=[
            jax.ShapeDtypeStruct((half, H2), jnp.bfloat16),
            jax.ShapeDtypeStruct((half, H2), jnp.bfloat16),
        ],
        scratch_shapes=[pltpu.VMEM((N, H1), jnp.bfloat16)],
    )(x, W1.astype(jnp.bfloat16), adj, adj, s1, sh1, W2.astype(jnp.bfloat16))

    mua, mub, outa, outb = pl.pallas_call(
        _pass2_kernel,
        grid=grid,
        in_specs=[
            pl.BlockSpec((BM, N), top),
            pl.BlockSpec((BM, N), bot),
            pl.BlockSpec((half, H2), const),
            pl.BlockSpec((half, H2), const),
            pl.BlockSpec((1, H2), const),
            pl.BlockSpec((1, H2), const),
            pl.BlockSpec((H2, C), const),
            pl.BlockSpec((1, C), const),
        ],
        out_specs=[
            pl.BlockSpec((BM, H2), top),
            pl.BlockSpec((BM, H2), top),
            pl.BlockSpec((BM, C), top),
            pl.BlockSpec((BM, C), top),
        ],
        out_shape=[
            jax.ShapeDtypeStruct((half, H2), jnp.float32),
            jax.ShapeDtypeStruct((half, H2), jnp.float32),
            jax.ShapeDtypeStruct((half, C), jnp.float32),
            jax.ShapeDtypeStruct((half, C), jnp.float32),
        ],
    )(adj, adj, ta, tb, s2, sh2, dwT, db)

    mu = jnp.concatenate([mua, mub], axis=0)
    out = jnp.concatenate([outa, outb], axis=0)
    return (out, mu, mu)


# speedup vs baseline: 1.0485x; 1.0485x over previous
import jax
import jax.numpy as jnp
from jax import lax
from jax.experimental import pallas as pl
from jax.experimental.pallas import tpu as pltpu

_EPS = 1e-5
_CH = 200
_NBUF = 4


def _ring_kernel(x_ref, w1_ref, s1_ref, sh1_ref, w2_ref, s2_ref, sh2_ref,
                 dw_ref, db_ref, adj_ref, mu_ref, out_ref,
                 bufs, support, t, sems):
    n = adj_ref.shape[0]
    nch = n // _CH
    total = 2 * nch

    def dma(chunk, slot):
        row = lax.rem(chunk, nch) * _CH
        return pltpu.make_async_copy(
            adj_ref.at[pl.ds(row, _CH), :], bufs.at[slot], sems.at[slot])

    for s in range(_NBUF):
        dma(s, s).start()
    support[...] = jnp.dot(
        x_ref[...].astype(jnp.bfloat16), w1_ref[...],
        preferred_element_type=jnp.float32).astype(jnp.bfloat16)

    def refill(chunk):
        @pl.when(chunk + _NBUF < total)
        def _():
            dma(chunk + _NBUF, lax.rem(chunk, _NBUF)).start()

    def body1(c, carry):
        slot = lax.rem(c, _NBUF)
        dma(c, slot).wait()
        acc = jnp.dot(bufs[slot].astype(jnp.bfloat16), support[...],
                      preferred_element_type=jnp.float32)
        h1 = jnp.maximum(acc * s1_ref[...] + sh1_ref[...], 0.0)
        t[pl.ds(c * _CH, _CH), :] = jnp.dot(
            h1.astype(jnp.bfloat16), w2_ref[...],
            preferred_element_type=jnp.float32).astype(jnp.bfloat16)
        refill(c)
        return carry

    lax.fori_loop(0, nch, body1, 0, unroll=False)

    def body2(c2, carry):
        c = c2 + nch
        slot = lax.rem(c, _NBUF)
        dma(c, slot).wait()
        acc = jnp.dot(bufs[slot].astype(jnp.bfloat16), t[...],
                      preferred_element_type=jnp.float32)
        mu = acc * s2_ref[...] + sh2_ref[...]
        mu_ref[pl.ds(c2 * _CH, _CH), :] = mu
        out_ref[pl.ds(c2 * _CH, _CH), :] = jnp.dot(
            mu, dw_ref[...], preferred_element_type=jnp.float32) + db_ref[...]
        refill(c)
        return carry

    lax.fori_loop(0, nch, body2, 0, unroll=False)


def kernel(x, adj, W1, W2, g1, b1, m1, v1, g2, b2, m2, v2, dec_W, dec_b):
    N, F = x.shape
    H1 = W1.shape[1]
    H2 = W2.shape[1]
    C = dec_W.shape[0]

    inv1 = g1 / jnp.sqrt(v1 + _EPS)
    s1 = inv1.reshape(1, H1)
    sh1 = (b1 - m1 * inv1).reshape(1, H1)
    inv2 = g2 / jnp.sqrt(v2 + _EPS)
    s2 = inv2.reshape(1, H2)
    sh2 = (b2 - m2 * inv2).reshape(1, H2)
    dwT = dec_W.T
    db = dec_b.reshape(1, C)

    vmem = pl.BlockSpec(memory_space=pltpu.MemorySpace.VMEM)
    mu, out = pl.pallas_call(
        _ring_kernel,
        in_specs=[
            vmem,
            vmem,
            vmem,
            vmem,
            vmem,
            vmem,
            vmem,
            vmem,
            vmem,
            pl.BlockSpec(memory_space=pltpu.MemorySpace.HBM),
        ],
        out_specs=[vmem, vmem],
        out_shape=[
            jax.ShapeDtypeStruct((N, H2), jnp.float32),
            jax.ShapeDtypeStruct((N, C), jnp.float32),
        ],
        scratch_shapes=[
            pltpu.VMEM((_NBUF, _CH, N), jnp.float32),
            pltpu.VMEM((N, H1), jnp.bfloat16),
            pltpu.VMEM((N, H2), jnp.bfloat16),
            pltpu.SemaphoreType.DMA((_NBUF,)),
        ],
    )(x, W1.astype(jnp.bfloat16), s1, sh1, W2.astype(jnp.bfloat16),
      s2, sh2, dwT, db, adj)

    return (out, mu, mu)


# speedup vs baseline: 1.0526x; 1.0039x over previous
import jax
import jax.numpy as jnp
from jax import lax
from jax.experimental import pallas as pl
from jax.experimental.pallas import tpu as pltpu

_EPS = 1e-5
_CH = 200
_NBUF = 4


def _ring_kernel(x_ref, w1_ref, s1_ref, sh1_ref, w2_ref, s2_ref, sh2_ref,
                 dw_ref, db_ref, adj_ref, mu_ref, out_ref,
                 bufs, support, t32, t16, sems):
    n = adj_ref.shape[0]
    nch = n // _CH
    total = 2 * nch

    def dma(chunk, slot):
        row = lax.rem(chunk, nch) * _CH
        return pltpu.make_async_copy(
            adj_ref.at[pl.ds(row, _CH), :], bufs.at[slot], sems.at[slot])

    for s in range(_NBUF):
        dma(s, s).start()
    support[...] = jnp.dot(
        x_ref[...].astype(jnp.bfloat16), w1_ref[...],
        preferred_element_type=jnp.float32).astype(jnp.bfloat16)

    def refill(chunk):
        @pl.when(chunk + _NBUF < total)
        def _():
            dma(chunk + _NBUF, lax.rem(chunk, _NBUF)).start()

    def body1(c, carry):
        slot = lax.rem(c, _NBUF)
        dma(c, slot).wait()
        acc = jnp.dot(bufs[slot].astype(jnp.bfloat16), support[...],
                      preferred_element_type=jnp.float32)
        h1 = jnp.maximum(acc * s1_ref[...] + sh1_ref[...], 0.0)
        t32[pl.ds(c * _CH, _CH), :] = jnp.dot(
            h1.astype(jnp.bfloat16), w2_ref[...],
            preferred_element_type=jnp.float32)
        refill(c)
        return carry

    lax.fori_loop(0, nch, body1, 0, unroll=False)
    t16[...] = t32[...].astype(jnp.bfloat16)

    def body2(c2, carry):
        c = c2 + nch
        slot = lax.rem(c, _NBUF)
        dma(c, slot).wait()
        acc = jnp.dot(bufs[slot].astype(jnp.bfloat16), t16[...],
                      preferred_element_type=jnp.float32)
        mu = acc * s2_ref[...] + sh2_ref[...]
        mu_ref[pl.ds(c2 * _CH, _CH), :] = mu
        out_ref[pl.ds(c2 * _CH, _CH), :] = jnp.dot(
            mu, dw_ref[...], preferred_element_type=jnp.float32) + db_ref[...]
        refill(c)
        return carry

    lax.fori_loop(0, nch, body2, 0, unroll=False)


def kernel(x, adj, W1, W2, g1, b1, m1, v1, g2, b2, m2, v2, dec_W, dec_b):
    N, F = x.shape
    H1 = W1.shape[1]
    H2 = W2.shape[1]
    C = dec_W.shape[0]

    inv1 = g1 / jnp.sqrt(v1 + _EPS)
    s1 = inv1.reshape(1, H1)
    sh1 = (b1 - m1 * inv1).reshape(1, H1)
    inv2 = g2 / jnp.sqrt(v2 + _EPS)
    s2 = inv2.reshape(1, H2)
    sh2 = (b2 - m2 * inv2).reshape(1, H2)
    dwT = dec_W.T
    db = dec_b.reshape(1, C)

    vmem = pl.BlockSpec(memory_space=pltpu.MemorySpace.VMEM)
    mu, out = pl.pallas_call(
        _ring_kernel,
        in_specs=[
            vmem,
            vmem,
            vmem,
            vmem,
            vmem,
            vmem,
            vmem,
            vmem,
            vmem,
            pl.BlockSpec(memory_space=pltpu.MemorySpace.HBM),
        ],
        out_specs=[vmem, vmem],
        out_shape=[
            jax.ShapeDtypeStruct((N, H2), jnp.float32),
            jax.ShapeDtypeStruct((N, C), jnp.float32),
        ],
        scratch_shapes=[
            pltpu.VMEM((_NBUF, _CH, N), jnp.float32),
            pltpu.VMEM((N, H1), jnp.bfloat16),
            pltpu.VMEM((N, H2), jnp.float32),
            pltpu.VMEM((N, H2), jnp.bfloat16),
            pltpu.SemaphoreType.DMA((_NBUF,)),
        ],
    )(x, W1.astype(jnp.bfloat16), s1, sh1, W2.astype(jnp.bfloat16),
      s2, sh2, dwT, db, adj)

    return (out, mu, mu)


# speedup vs baseline: 1.0649x; 1.0117x over previous
import jax
import jax.numpy as jnp
from jax import lax
from jax.experimental import pallas as pl
from jax.experimental.pallas import tpu as pltpu

_EPS = 1e-5
_CH = 200
_NBUF = 5


def _ring_kernel(x_ref, w1_ref, s1_ref, sh1_ref, w2_ref, s2_ref, sh2_ref,
                 dw_ref, db_ref, adj_ref, mu_ref, out_ref,
                 bufs, support, t32, t16, sems):
    n = adj_ref.shape[0]
    nch = n // _CH
    total = 2 * nch

    def dma(chunk, slot):
        row = lax.rem(chunk, nch) * _CH
        return pltpu.make_async_copy(
            adj_ref.at[pl.ds(row, _CH), :], bufs.at[slot], sems.at[slot])

    for s in range(_NBUF):
        dma(s, s).start()
    support[...] = jnp.dot(
        x_ref[...].astype(jnp.bfloat16), w1_ref[...],
        preferred_element_type=jnp.float32).astype(jnp.bfloat16)

    def refill(chunk):
        @pl.when(chunk + _NBUF < total)
        def _():
            dma(chunk + _NBUF, lax.rem(chunk, _NBUF)).start()

    def body1(c, carry):
        slot = lax.rem(c, _NBUF)
        dma(c, slot).wait()
        acc = jnp.dot(bufs[slot].astype(jnp.bfloat16), support[...],
                      preferred_element_type=jnp.float32)
        h1 = jnp.maximum(acc * s1_ref[...] + sh1_ref[...], 0.0)
        t32[pl.ds(c * _CH, _CH), :] = jnp.dot(
            h1.astype(jnp.bfloat16), w2_ref[...],
            preferred_element_type=jnp.float32)
        refill(c)
        return carry

    lax.fori_loop(0, nch, body1, 0, unroll=False)
    t16[...] = t32[...].astype(jnp.bfloat16)

    def body2(c2, carry):
        c = c2 + nch
        slot = lax.rem(c, _NBUF)
        dma(c, slot).wait()
        acc = jnp.dot(bufs[slot].astype(jnp.bfloat16), t16[...],
                      preferred_element_type=jnp.float32)
        mu = acc * s2_ref[...] + sh2_ref[...]
        mu_ref[pl.ds(c2 * _CH, _CH), :] = mu
        refill(c)
        return carry

    lax.fori_loop(0, nch, body2, 0, unroll=False)
    out_ref[...] = lax.dot_general(
        dw_ref[...], mu_ref[...], (((0,), (1,)), ((), ())),
        preferred_element_type=jnp.float32) + db_ref[...].reshape(-1, 1)


def kernel(x, adj, W1, W2, g1, b1, m1, v1, g2, b2, m2, v2, dec_W, dec_b):
    N, F = x.shape
    H1 = W1.shape[1]
    H2 = W2.shape[1]
    C = dec_W.shape[0]

    inv1 = g1 / jnp.sqrt(v1 + _EPS)
    s1 = inv1.reshape(1, H1)
    sh1 = (b1 - m1 * inv1).reshape(1, H1)
    inv2 = g2 / jnp.sqrt(v2 + _EPS)
    s2 = inv2.reshape(1, H2)
    sh2 = (b2 - m2 * inv2).reshape(1, H2)
    dwT = dec_W.T
    db = dec_b.reshape(1, C)

    vmem = pl.BlockSpec(memory_space=pltpu.MemorySpace.VMEM)
    mu, out = pl.pallas_call(
        _ring_kernel,
        in_specs=[
            vmem,
            vmem,
            vmem,
            vmem,
            vmem,
            vmem,
            vmem,
            vmem,
            vmem,
            pl.BlockSpec(memory_space=pltpu.MemorySpace.HBM),
        ],
        out_specs=[vmem, vmem],
        out_shape=[
            jax.ShapeDtypeStruct((N, H2), jnp.float32),
            jax.ShapeDtypeStruct((C, N), jnp.float32),
        ],
        scratch_shapes=[
            pltpu.VMEM((_NBUF, _CH, N), jnp.float32),
            pltpu.VMEM((N, H1), jnp.bfloat16),
            pltpu.VMEM((N, H2), jnp.float32),
            pltpu.VMEM((N, H2), jnp.bfloat16),
            pltpu.SemaphoreType.DMA((_NBUF,)),
        ],
    )(x, W1.astype(jnp.bfloat16), s1, sh1, W2.astype(jnp.bfloat16),
      s2, sh2, dwT, db, adj)

    out = out.reshape(N, C)
    return (out, mu, mu)


# speedup vs baseline: 1.0853x; 1.0191x over previous
import jax
import jax.numpy as jnp
from jax import lax
from jax.experimental import pallas as pl
from jax.experimental.pallas import tpu as pltpu

_EPS = 1e-5
_CH = 200
_NBUF = 5


def _ring_kernel(x_ref, w1_ref, w2_ref, g1_ref, b1_ref, m1_ref, v1_ref,
                 g2_ref, b2_ref, m2_ref, v2_ref, dw_ref, db_ref, adj_ref,
                 mu_ref, out_ref, bufs, support, t32, t16, sems):
    n = adj_ref.shape[0]
    nch = n // _CH
    total = 2 * nch

    def dma(chunk, slot):
        row = lax.rem(chunk, nch) * _CH
        return pltpu.make_async_copy(
            adj_ref.at[pl.ds(row, _CH), :], bufs.at[slot], sems.at[slot])

    for s in range(_NBUF):
        dma(s, s).start()

    inv1 = g1_ref[...] * lax.rsqrt(v1_ref[...] + _EPS)
    s1 = inv1
    sh1 = b1_ref[...] - m1_ref[...] * inv1
    inv2 = g2_ref[...] * lax.rsqrt(v2_ref[...] + _EPS)
    s2 = inv2
    sh2 = b2_ref[...] - m2_ref[...] * inv2

    w2 = w2_ref[...].astype(jnp.bfloat16)
    support[...] = jnp.dot(
        x_ref[...].astype(jnp.bfloat16), w1_ref[...].astype(jnp.bfloat16),
        preferred_element_type=jnp.float32).astype(jnp.bfloat16)

    def refill(chunk):
        @pl.when(chunk + _NBUF < total)
        def _():
            dma(chunk + _NBUF, lax.rem(chunk, _NBUF)).start()

    def body1(c, carry):
        slot = lax.rem(c, _NBUF)
        dma(c, slot).wait()
        acc = jnp.dot(bufs[slot].astype(jnp.bfloat16), support[...],
                      preferred_element_type=jnp.float32)
        h1 = jnp.maximum(acc * s1 + sh1, 0.0)
        t32[pl.ds(c * _CH, _CH), :] = jnp.dot(
            h1.astype(jnp.bfloat16), w2, preferred_element_type=jnp.float32)
        refill(c)
        return carry

    lax.fori_loop(0, nch, body1, 0, unroll=False)
    t16[...] = t32[...].astype(jnp.bfloat16)

    def body2(c2, carry):
        c = c2 + nch
        slot = lax.rem(c, _NBUF)
        dma(c, slot).wait()
        acc = jnp.dot(bufs[slot].astype(jnp.bfloat16), t16[...],
                      preferred_element_type=jnp.float32)
        mu_ref[pl.ds(c2 * _CH, _CH), :] = acc * s2 + sh2
        refill(c)
        return carry

    lax.fori_loop(0, nch, body2, 0, unroll=False)
    out_ref[...] = lax.dot_general(
        dw_ref[...], mu_ref[...], (((1,), (1,)), ((), ())),
        preferred_element_type=jnp.float32) + db_ref[...].reshape(-1, 1)


def kernel(x, adj, W1, W2, g1, b1, m1, v1, g2, b2, m2, v2, dec_W, dec_b):
    N, F = x.shape
    H1 = W1.shape[1]
    H2 = W2.shape[1]
    C = dec_W.shape[0]

    vmem = pl.BlockSpec(memory_space=pltpu.MemorySpace.VMEM)
    mu, out = pl.pallas_call(
        _ring_kernel,
        in_specs=[vmem] * 13 + [pl.BlockSpec(memory_space=pltpu.MemorySpace.HBM)],
        out_specs=[vmem, vmem],
        out_shape=[
            jax.ShapeDtypeStruct((N, H2), jnp.float32),
            jax.ShapeDtypeStruct((C, N), jnp.float32),
        ],
        scratch_shapes=[
            pltpu.VMEM((_NBUF, _CH, N), jnp.float32),
            pltpu.VMEM((N, H1), jnp.bfloat16),
            pltpu.VMEM((N, H2), jnp.float32),
            pltpu.VMEM((N, H2), jnp.bfloat16),
            pltpu.SemaphoreType.DMA((_NBUF,)),
        ],
    )(x, W1, W2, g1, b1, m1, v1, g2, b2, m2, v2, dec_W, dec_b, adj)

    out = out.reshape(N, C)
    return (out, mu, mu)


# speedup vs baseline: 1.0855x; 1.0002x over previous
import jax
import jax.numpy as jnp
from jax import lax
from jax.experimental import pallas as pl
from jax.experimental.pallas import tpu as pltpu

_EPS = 1e-5
_CH = 80
_NBUF = 12


def _ring_kernel(x_ref, w1_ref, w2_ref, g1_ref, b1_ref, m1_ref, v1_ref,
                 g2_ref, b2_ref, m2_ref, v2_ref, dw_ref, db_ref, adj_ref,
                 mu_ref, out_ref, bufs, support, t32, t16, sems):
    n = adj_ref.shape[0]
    nch = n // _CH
    total = 2 * nch

    def dma(chunk, slot):
        row = lax.rem(chunk, nch) * _CH
        return pltpu.make_async_copy(
            adj_ref.at[pl.ds(row, _CH), :], bufs.at[slot], sems.at[slot])

    for s in range(_NBUF):
        dma(s, s).start()

    inv1 = g1_ref[...] * lax.rsqrt(v1_ref[...] + _EPS)
    s1 = inv1
    sh1 = b1_ref[...] - m1_ref[...] * inv1
    inv2 = g2_ref[...] * lax.rsqrt(v2_ref[...] + _EPS)
    s2 = inv2
    sh2 = b2_ref[...] - m2_ref[...] * inv2

    w2 = w2_ref[...].astype(jnp.bfloat16)
    support[...] = jnp.dot(
        x_ref[...].astype(jnp.bfloat16), w1_ref[...].astype(jnp.bfloat16),
        preferred_element_type=jnp.float32).astype(jnp.bfloat16)

    def refill(chunk):
        @pl.when(chunk + _NBUF < total)
        def _():
            dma(chunk + _NBUF, lax.rem(chunk, _NBUF)).start()

    def body1(c, carry):
        slot = lax.rem(c, _NBUF)
        dma(c, slot).wait()
        acc = jnp.dot(bufs[slot].astype(jnp.bfloat16), support[...],
                      preferred_element_type=jnp.float32)
        h1 = jnp.maximum(acc * s1 + sh1, 0.0)
        t32[pl.ds(c * _CH, _CH), :] = jnp.dot(
            h1.astype(jnp.bfloat16), w2, preferred_element_type=jnp.float32)
        refill(c)
        return carry

    lax.fori_loop(0, nch, body1, 0, unroll=False)
    t16[...] = t32[...].astype(jnp.bfloat16)

    def body2(c2, carry):
        c = c2 + nch
        slot = lax.rem(c, _NBUF)
        dma(c, slot).wait()
        acc = jnp.dot(bufs[slot].astype(jnp.bfloat16), t16[...],
                      preferred_element_type=jnp.float32)
        mu_ref[pl.ds(c2 * _CH, _CH), :] = acc * s2 + sh2
        refill(c)
        return carry

    lax.fori_loop(0, nch, body2, 0, unroll=False)
    out_ref[...] = lax.dot_general(
        dw_ref[...], mu_ref[...], (((1,), (1,)), ((), ())),
        preferred_element_type=jnp.float32) + db_ref[...].reshape(-1, 1)


def kernel(x, adj, W1, W2, g1, b1, m1, v1, g2, b2, m2, v2, dec_W, dec_b):
    N, F = x.shape
    H1 = W1.shape[1]
    H2 = W2.shape[1]
    C = dec_W.shape[0]

    vmem = pl.BlockSpec(memory_space=pltpu.MemorySpace.VMEM)
    mu, out = pl.pallas_call(
        _ring_kernel,
        in_specs=[vmem] * 13 + [pl.BlockSpec(memory_space=pltpu.MemorySpace.HBM)],
        out_specs=[vmem, vmem],
        out_shape=[
            jax.ShapeDtypeStruct((N, H2), jnp.float32),
            jax.ShapeDtypeStruct((C, N), jnp.float32),
        ],
        scratch_shapes=[
            pltpu.VMEM((_NBUF, _CH, N), jnp.float32),
            pltpu.VMEM((N, H1), jnp.bfloat16),
            pltpu.VMEM((N, H2), jnp.float32),
            pltpu.VMEM((N, H2), jnp.bfloat16),
            pltpu.SemaphoreType.DMA((_NBUF,)),
        ],
    )(x, W1, W2, g1, b1, m1, v1, g2, b2, m2, v2, dec_W, dec_b, adj)

    out = out.reshape(N, C)
    return (out, mu, mu)
